# baseline (speedup 1.0000x reference)
import functools
import math

import jax
import jax.numpy as jnp
from jax import lax
from jax.experimental import pallas as pl
from jax.experimental.pallas import tpu as pltpu
from jax.experimental.pallas import tpu_sc as plsc

_EPS = 1e-5
_T = 256


def _sc_gather_rows(table, idx, chunk):
    V, D = table.shape
    (Bn,) = idx.shape
    info = plsc.get_sparse_core_info()
    NW = info.num_cores * info.num_subcores
    assert Bn % (NW * chunk) == 0
    b_per_w = Bn // NW
    nchunks = b_per_w // chunk
    mesh = plsc.VectorSubcoreMesh(core_axis_name="c", subcore_axis_name="s")

    @functools.partial(
        pl.kernel,
        mesh=mesh,
        out_type=jax.ShapeDtypeStruct((Bn, D), table.dtype),
        scratch_types=[
            pltpu.VMEM((chunk,), jnp.int32),
            pltpu.VMEM((chunk, D), table.dtype),
            pltpu.SemaphoreType.DMA,
        ],
    )
    def k(table_hbm, idx_hbm, out_hbm, idx_v, rows_v, sem):
        wid = lax.axis_index("s") * info.num_cores + lax.axis_index("c")
        base = wid * b_per_w
        for c in range(nchunks):
            off = base + c * chunk
            pltpu.sync_copy(idx_hbm.at[pl.ds(off, chunk)], idx_v)
            pltpu.async_copy(table_hbm.at[idx_v], rows_v, sem).wait()
            pltpu.sync_copy(rows_v, out_hbm.at[pl.ds(off, chunk)])

    return k(table, idx)


def _sc_scatter_rows(x, idx, n_out, chunk):
    N, D = x.shape
    info = plsc.get_sparse_core_info()
    NW = info.num_cores * info.num_subcores
    assert N % (NW * chunk) == 0
    b_per_w = N // NW
    nchunks = b_per_w // chunk
    mesh = plsc.VectorSubcoreMesh(core_axis_name="c", subcore_axis_name="s")

    @functools.partial(
        pl.kernel,
        mesh=mesh,
        out_type=jax.ShapeDtypeStruct((n_out, D), x.dtype),
        scratch_types=[
            pltpu.VMEM((chunk,), jnp.int32),
            pltpu.VMEM((chunk, D), x.dtype),
            pltpu.SemaphoreType.DMA,
        ],
    )
    def k(x_hbm, idx_hbm, out_hbm, idx_v, rows_v, sem):
        wid = lax.axis_index("s") * info.num_cores + lax.axis_index("c")
        base = wid * b_per_w
        for c in range(nchunks):
            off = base + c * chunk
            pltpu.sync_copy(idx_hbm.at[pl.ds(off, chunk)], idx_v)
            pltpu.sync_copy(x_hbm.at[pl.ds(off, chunk)], rows_v)
            pltpu.async_copy(rows_v, out_hbm.at[idx_v], sem).wait()

    return k(x, idx)


_FB = 1024


def _ffn_block_kernel(be_ref, x_ref, w1_ref, b1_ref, w2_ref, b2_ref,
                      g_ref, bt_ref, o_ref, acc_ref, w1c_ref, w2c_ref, *, nf):
    f = pl.program_id(0)
    g = pl.program_id(1)
    sl = pl.ds(g * _T, _T)

    new_w = jnp.logical_or(
        g == 0, be_ref[g] != be_ref[jnp.maximum(g - 1, 0)])

    @pl.when(new_w)
    def _():
        w1c_ref[...] = w1_ref[0].astype(jnp.bfloat16)
        w2c_ref[...] = w2_ref[0].astype(jnp.bfloat16)

    x = x_ref[...]
    h = jnp.dot(x.astype(jnp.bfloat16), w1c_ref[...],
                preferred_element_type=jnp.float32)
    h = h + b1_ref[0]
    h = 0.5 * h * (1.0 + lax.erf(h * (1.0 / math.sqrt(2.0))))
    y = jnp.dot(h.astype(jnp.bfloat16), w2c_ref[...],
                preferred_element_type=jnp.float32)

    @pl.when(f == 0)
    def _():
        acc_ref[sl, :] = y + x + b2_ref[0]

    @pl.when(f > 0)
    def _():
        acc_ref[sl, :] += y

    @pl.when(f == nf - 1)
    def _():
        yv = acc_ref[sl, :]
        mean = jnp.mean(yv, axis=1, keepdims=True)
        yc = yv - mean
        var = jnp.mean(yc * yc, axis=1, keepdims=True)
        o_ref[...] = yc * lax.rsqrt(var + _EPS) * g_ref[...] + bt_ref[...]


def _grouped_ffn(xs, W1, b1, W2, b2, gamma2d, beta2d, block_expert, G):
    N_PAD, D = xs.shape
    E, _, F = W1.shape
    nf = F // _FB
    grid_spec = pltpu.PrefetchScalarGridSpec(
        num_scalar_prefetch=1,
        grid=(nf, G),
        in_specs=[
            pl.BlockSpec((_T, D), lambda f, g, be: (g, 0)),
            pl.BlockSpec((1, D, _FB), lambda f, g, be: (be[g], 0, f)),
            pl.BlockSpec((1, 1, _FB), lambda f, g, be: (be[g], 0, f)),
            pl.BlockSpec((1, _FB, D), lambda f, g, be: (be[g], f, 0)),
            pl.BlockSpec((1, 1, D), lambda f, g, be: (be[g], 0, 0)),
            pl.BlockSpec((1, D), lambda f, g, be: (0, 0)),
            pl.BlockSpec((1, D), lambda f, g, be: (0, 0)),
        ],
        out_specs=pl.BlockSpec(
            (_T, D), lambda f, g, be: (jnp.where(f == nf - 1, g, 0), 0)),
        scratch_shapes=[pltpu.VMEM((N_PAD, D), jnp.float32),
                        pltpu.VMEM((D, _FB), jnp.bfloat16),
                        pltpu.VMEM((_FB, D), jnp.bfloat16)],
    )
    return pl.pallas_call(
        functools.partial(_ffn_block_kernel, nf=nf),
        grid_spec=grid_spec,
        out_shape=jax.ShapeDtypeStruct((N_PAD, D), jnp.float32),
    )(block_expert, xs, W1, b1, W2, b2, gamma2d, beta2d)


def kernel(hidden_states, W1, b1, W2, b2, gamma, beta, expert_assign):
    B, S, D = hidden_states.shape
    E, _, F = W1.shape
    N = B * S
    G = N // _T + E
    N_PAD = G * _T

    x = hidden_states.reshape(N, D)
    e = expert_assign

    oh = (e[:, None] == jnp.arange(E, dtype=e.dtype)[None, :]).astype(jnp.int32)
    csum = jnp.cumsum(oh, axis=0)
    counts = csum[-1]
    rank = jnp.take_along_axis(csum, e[:, None], axis=1)[:, 0] - 1
    blocks_e = (counts + _T - 1) // _T
    block_start = jnp.concatenate(
        [jnp.zeros((1,), jnp.int32), jnp.cumsum(blocks_e).astype(jnp.int32)])
    dest = block_start[e] * _T + rank
    gidx = jnp.arange(G, dtype=jnp.int32)
    block_expert = jnp.minimum(
        jnp.sum(block_start[1:E + 1][None, :] <= gidx[:, None], axis=1),
        E - 1).astype(jnp.int32)

    xs = _sc_scatter_rows(x, dest, N_PAD, chunk=64)

    zs = _grouped_ffn(xs, W1, b1.reshape(E, 1, F), W2, b2.reshape(E, 1, D),
                      gamma.reshape(1, D), beta.reshape(1, D),
                      block_expert, block_start[E])

    out = _sc_gather_rows(zs, dest, chunk=64)
    return out.reshape(B, S, D)


# speedup vs baseline: 1.0867x; 1.0867x over previous
import functools
import math

import jax
import jax.numpy as jnp
from jax import lax
from jax.experimental import pallas as pl
from jax.experimental.pallas import tpu as pltpu
from jax.experimental.pallas import tpu_sc as plsc

_EPS = 1e-5
_T = 256


def _sc_gather_rows(table, idx, chunk):
    V, D = table.shape
    (Bn,) = idx.shape
    info = plsc.get_sparse_core_info()
    NW = info.num_cores * info.num_subcores
    assert Bn % (NW * chunk) == 0
    b_per_w = Bn // NW
    nchunks = b_per_w // chunk
    mesh = plsc.VectorSubcoreMesh(core_axis_name="c", subcore_axis_name="s")

    @functools.partial(
        pl.kernel,
        mesh=mesh,
        out_type=jax.ShapeDtypeStruct((Bn, D), table.dtype),
        scratch_types=[
            pltpu.VMEM((chunk,), jnp.int32),
            pltpu.VMEM((chunk, D), table.dtype),
            pltpu.SemaphoreType.DMA,
        ],
    )
    def k(table_hbm, idx_hbm, out_hbm, idx_v, rows_v, sem):
        wid = lax.axis_index("s") * info.num_cores + lax.axis_index("c")
        base = wid * b_per_w
        for c in range(nchunks):
            off = base + c * chunk
            pltpu.sync_copy(idx_hbm.at[pl.ds(off, chunk)], idx_v)
            pltpu.async_copy(table_hbm.at[idx_v], rows_v, sem).wait()
            pltpu.sync_copy(rows_v, out_hbm.at[pl.ds(off, chunk)])

    return k(table, idx)


def _sc_scatter_rows(x, idx, n_out, chunk):
    N, D = x.shape
    info = plsc.get_sparse_core_info()
    NW = info.num_cores * info.num_subcores
    assert N % (NW * chunk) == 0
    b_per_w = N // NW
    nchunks = b_per_w // chunk
    mesh = plsc.VectorSubcoreMesh(core_axis_name="c", subcore_axis_name="s")

    @functools.partial(
        pl.kernel,
        mesh=mesh,
        out_type=jax.ShapeDtypeStruct((n_out, D), x.dtype),
        scratch_types=[
            pltpu.VMEM((chunk,), jnp.int32),
            pltpu.VMEM((chunk, D), x.dtype),
            pltpu.SemaphoreType.DMA,
        ],
    )
    def k(x_hbm, idx_hbm, out_hbm, idx_v, rows_v, sem):
        wid = lax.axis_index("s") * info.num_cores + lax.axis_index("c")
        base = wid * b_per_w
        for c in range(nchunks):
            off = base + c * chunk
            pltpu.sync_copy(idx_hbm.at[pl.ds(off, chunk)], idx_v)
            pltpu.sync_copy(x_hbm.at[pl.ds(off, chunk)], rows_v)
            pltpu.async_copy(rows_v, out_hbm.at[idx_v], sem).wait()

    return k(x, idx)


_FB = 1024


def _ffn_block_kernel(be_ref, x_ref, w1_ref, b1_ref, w2_ref, b2_ref,
                      g_ref, bt_ref, o_ref, acc_ref, *, nf):
    f = pl.program_id(0)
    g = pl.program_id(1)
    sl = pl.ds(g * _T, _T)
    x = x_ref[...]
    h = jnp.dot(x, w1_ref[0], preferred_element_type=jnp.float32)
    h = h + b1_ref[0]
    h = 0.5 * h * (1.0 + lax.erf(h * (1.0 / math.sqrt(2.0))))
    y = jnp.dot(h, w2_ref[0], preferred_element_type=jnp.float32)

    @pl.when(f == 0)
    def _():
        acc_ref[sl, :] = y + x + b2_ref[0]

    @pl.when(f > 0)
    def _():
        acc_ref[sl, :] += y

    @pl.when(f == nf - 1)
    def _():
        yv = acc_ref[sl, :]
        mean = jnp.mean(yv, axis=1, keepdims=True)
        yc = yv - mean
        var = jnp.mean(yc * yc, axis=1, keepdims=True)
        o_ref[...] = yc * lax.rsqrt(var + _EPS) * g_ref[...] + bt_ref[...]


def _grouped_ffn(xs, W1, b1, W2, b2, gamma2d, beta2d, block_expert, G):
    N_PAD, D = xs.shape
    E, _, F = W1.shape
    nf = F // _FB
    grid_spec = pltpu.PrefetchScalarGridSpec(
        num_scalar_prefetch=1,
        grid=(nf, G),
        in_specs=[
            pl.BlockSpec((_T, D), lambda f, g, be: (g, 0)),
            pl.BlockSpec((1, D, _FB), lambda f, g, be: (be[g], 0, f)),
            pl.BlockSpec((1, 1, _FB), lambda f, g, be: (be[g], 0, f)),
            pl.BlockSpec((1, _FB, D), lambda f, g, be: (be[g], f, 0)),
            pl.BlockSpec((1, 1, D), lambda f, g, be: (be[g], 0, 0)),
            pl.BlockSpec((1, D), lambda f, g, be: (0, 0)),
            pl.BlockSpec((1, D), lambda f, g, be: (0, 0)),
        ],
        out_specs=pl.BlockSpec(
            (_T, D), lambda f, g, be: (jnp.where(f == nf - 1, g, 0), 0)),
        scratch_shapes=[pltpu.VMEM((N_PAD, D), jnp.float32)],
    )
    return pl.pallas_call(
        functools.partial(_ffn_block_kernel, nf=nf),
        grid_spec=grid_spec,
        out_shape=jax.ShapeDtypeStruct((N_PAD, D), jnp.float32),
    )(block_expert, xs, W1, b1, W2, b2, gamma2d, beta2d)


def kernel(hidden_states, W1, b1, W2, b2, gamma, beta, expert_assign):
    B, S, D = hidden_states.shape
    E, _, F = W1.shape
    N = B * S
    G = N // _T + E
    N_PAD = G * _T

    x = hidden_states.reshape(N, D)
    e = expert_assign

    oh = (e[:, None] == jnp.arange(E, dtype=e.dtype)[None, :]).astype(jnp.int32)
    csum = jnp.cumsum(oh, axis=0)
    counts = csum[-1]
    rank = jnp.sum(csum * oh, axis=1) - 1
    blocks_e = (counts + _T - 1) // _T
    block_start = jnp.concatenate(
        [jnp.zeros((1,), jnp.int32), jnp.cumsum(blocks_e).astype(jnp.int32)])
    dest = block_start[e] * _T + rank
    gidx = jnp.arange(G, dtype=jnp.int32)
    block_expert = jnp.minimum(
        jnp.sum(block_start[1:E + 1][None, :] <= gidx[:, None], axis=1),
        E - 1).astype(jnp.int32)

    xs = _sc_scatter_rows(x, dest, N_PAD, chunk=64)

    zs = _grouped_ffn(xs, W1, b1.reshape(E, 1, F), W2, b2.reshape(E, 1, D),
                      gamma.reshape(1, D), beta.reshape(1, D),
                      block_expert, block_start[E])

    out = _sc_gather_rows(zs, dest, chunk=64)
    return out.reshape(B, S, D)


# speedup vs baseline: 1.2337x; 1.1352x over previous
import functools
import math

import jax
import jax.numpy as jnp
from jax import lax
from jax.experimental import pallas as pl
from jax.experimental.pallas import tpu as pltpu
from jax.experimental.pallas import tpu_sc as plsc

_EPS = 1e-5
_T = 256


def _sc_gather_rows(table, idx, chunk):
    V, D = table.shape
    (Bn,) = idx.shape
    info = plsc.get_sparse_core_info()
    NW = info.num_cores * info.num_subcores
    assert Bn % (NW * chunk) == 0
    b_per_w = Bn // NW
    nchunks = b_per_w // chunk
    mesh = plsc.VectorSubcoreMesh(core_axis_name="c", subcore_axis_name="s")

    @functools.partial(
        pl.kernel,
        mesh=mesh,
        out_type=jax.ShapeDtypeStruct((Bn, D), table.dtype),
        scratch_types=[
            pltpu.VMEM((chunk,), jnp.int32),
            pltpu.VMEM((chunk, D), table.dtype),
            pltpu.SemaphoreType.DMA,
        ],
    )
    def k(table_hbm, idx_hbm, out_hbm, idx_v, rows_v, sem):
        wid = lax.axis_index("s") * info.num_cores + lax.axis_index("c")
        base = wid * b_per_w
        for c in range(nchunks):
            off = base + c * chunk
            pltpu.sync_copy(idx_hbm.at[pl.ds(off, chunk)], idx_v)
            pltpu.async_copy(table_hbm.at[idx_v], rows_v, sem).wait()
            pltpu.sync_copy(rows_v, out_hbm.at[pl.ds(off, chunk)])

    return k(table, idx)


def _sc_scatter_rows(x, idx, n_out, chunk):
    N, D = x.shape
    info = plsc.get_sparse_core_info()
    NW = info.num_cores * info.num_subcores
    assert N % (NW * chunk) == 0
    b_per_w = N // NW
    nchunks = b_per_w // chunk
    mesh = plsc.VectorSubcoreMesh(core_axis_name="c", subcore_axis_name="s")

    @functools.partial(
        pl.kernel,
        mesh=mesh,
        out_type=jax.ShapeDtypeStruct((n_out, D), x.dtype),
        scratch_types=[
            pltpu.VMEM((chunk,), jnp.int32),
            pltpu.VMEM((chunk, D), x.dtype),
            pltpu.SemaphoreType.DMA,
        ],
    )
    def k(x_hbm, idx_hbm, out_hbm, idx_v, rows_v, sem):
        wid = lax.axis_index("s") * info.num_cores + lax.axis_index("c")
        base = wid * b_per_w
        for c in range(nchunks):
            off = base + c * chunk
            pltpu.sync_copy(idx_hbm.at[pl.ds(off, chunk)], idx_v)
            pltpu.sync_copy(x_hbm.at[pl.ds(off, chunk)], rows_v)
            pltpu.async_copy(rows_v, out_hbm.at[idx_v], sem).wait()

    return k(x, idx)


_FB = 2048


def _ffn_block_kernel(be_ref, x_ref, w1_ref, b1_ref, w2_ref, b2_ref,
                      g_ref, bt_ref, o_ref, acc_ref, *, nf):
    f = pl.program_id(0)
    g = pl.program_id(1)
    sl = pl.ds(g * _T, _T)
    x = x_ref[...]
    h = jnp.dot(x, w1_ref[0], preferred_element_type=jnp.float32)
    h = h + b1_ref[0]
    h = 0.5 * h * (1.0 + lax.erf(h * (1.0 / math.sqrt(2.0))))
    y = jnp.dot(h, w2_ref[0], preferred_element_type=jnp.float32)

    @pl.when(f == 0)
    def _():
        acc_ref[sl, :] = (y + x + b2_ref[0]).astype(jnp.bfloat16)

    @pl.when(f > 0)
    def _():
        acc_ref[sl, :] += y.astype(jnp.bfloat16)

    @pl.when(f == nf - 1)
    def _():
        yv = acc_ref[sl, :].astype(jnp.float32)
        mean = jnp.mean(yv, axis=1, keepdims=True)
        yc = yv - mean
        var = jnp.mean(yc * yc, axis=1, keepdims=True)
        o_ref[...] = yc * lax.rsqrt(var + _EPS) * g_ref[...] + bt_ref[...]


def _grouped_ffn(xs, W1, b1, W2, b2, gamma2d, beta2d, block_expert, G):
    N_PAD, D = xs.shape
    E, _, F = W1.shape
    nf = F // _FB
    grid_spec = pltpu.PrefetchScalarGridSpec(
        num_scalar_prefetch=1,
        grid=(nf, G),
        in_specs=[
            pl.BlockSpec((_T, D), lambda f, g, be: (g, 0)),
            pl.BlockSpec((1, D, _FB), lambda f, g, be: (be[g], 0, f)),
            pl.BlockSpec((1, 1, _FB), lambda f, g, be: (be[g], 0, f)),
            pl.BlockSpec((1, _FB, D), lambda f, g, be: (be[g], f, 0)),
            pl.BlockSpec((1, 1, D), lambda f, g, be: (be[g], 0, 0)),
            pl.BlockSpec((1, D), lambda f, g, be: (0, 0)),
            pl.BlockSpec((1, D), lambda f, g, be: (0, 0)),
        ],
        out_specs=pl.BlockSpec(
            (_T, D), lambda f, g, be: (jnp.where(f == nf - 1, g, 0), 0)),
        scratch_shapes=[pltpu.VMEM((N_PAD, D), jnp.bfloat16)],
    )
    return pl.pallas_call(
        functools.partial(_ffn_block_kernel, nf=nf),
        grid_spec=grid_spec,
        out_shape=jax.ShapeDtypeStruct((N_PAD, D), jnp.float32),
    )(block_expert, xs, W1, b1, W2, b2, gamma2d, beta2d)


def kernel(hidden_states, W1, b1, W2, b2, gamma, beta, expert_assign):
    B, S, D = hidden_states.shape
    E, _, F = W1.shape
    N = B * S
    G = N // _T + E
    N_PAD = G * _T

    x = hidden_states.reshape(N, D)
    e = expert_assign

    oh = (e[:, None] == jnp.arange(E, dtype=e.dtype)[None, :]).astype(jnp.int32)
    csum = jnp.cumsum(oh, axis=0)
    counts = csum[-1]
    rank = jnp.sum(csum * oh, axis=1) - 1
    blocks_e = (counts + _T - 1) // _T
    block_start = jnp.concatenate(
        [jnp.zeros((1,), jnp.int32), jnp.cumsum(blocks_e).astype(jnp.int32)])
    dest = block_start[e] * _T + rank
    gidx = jnp.arange(G, dtype=jnp.int32)
    block_expert = jnp.minimum(
        jnp.sum(block_start[1:E + 1][None, :] <= gidx[:, None], axis=1),
        E - 1).astype(jnp.int32)

    xs = _sc_scatter_rows(x, dest, N_PAD, chunk=64)

    zs = _grouped_ffn(xs, W1, b1.reshape(E, 1, F), W2, b2.reshape(E, 1, D),
                      gamma.reshape(1, D), beta.reshape(1, D),
                      block_expert, block_start[E])

    out = _sc_gather_rows(zs, dest, chunk=64)
    return out.reshape(B, S, D)


# speedup vs baseline: 1.2442x; 1.0085x over previous
import functools
import math

import jax
import jax.numpy as jnp
from jax import lax
from jax.experimental import pallas as pl
from jax.experimental.pallas import tpu as pltpu
from jax.experimental.pallas import tpu_sc as plsc

_EPS = 1e-5
_T = 256


def _sc_gather_rows(table, idx, chunk):
    V, D = table.shape
    (Bn,) = idx.shape
    info = plsc.get_sparse_core_info()
    NW = info.num_cores * info.num_subcores
    assert Bn % (NW * chunk) == 0
    b_per_w = Bn // NW
    nchunks = b_per_w // chunk
    mesh = plsc.VectorSubcoreMesh(core_axis_name="c", subcore_axis_name="s")

    @functools.partial(
        pl.kernel,
        mesh=mesh,
        out_type=jax.ShapeDtypeStruct((Bn, D), table.dtype),
        scratch_types=[
            pltpu.VMEM((chunk,), jnp.int32),
            pltpu.VMEM((chunk, D), table.dtype),
            pltpu.SemaphoreType.DMA,
        ],
    )
    def k(table_hbm, idx_hbm, out_hbm, idx_v, rows_v, sem):
        wid = lax.axis_index("s") * info.num_cores + lax.axis_index("c")
        base = wid * b_per_w
        for c in range(nchunks):
            off = base + c * chunk
            pltpu.sync_copy(idx_hbm.at[pl.ds(off, chunk)], idx_v)
            pltpu.async_copy(table_hbm.at[idx_v], rows_v, sem).wait()
            pltpu.sync_copy(rows_v, out_hbm.at[pl.ds(off, chunk)])

    return k(table, idx)


def _sc_scatter_rows(x, idx, n_out, chunk):
    N, D = x.shape
    info = plsc.get_sparse_core_info()
    NW = info.num_cores * info.num_subcores
    assert N % (NW * chunk) == 0
    b_per_w = N // NW
    nchunks = b_per_w // chunk
    mesh = plsc.VectorSubcoreMesh(core_axis_name="c", subcore_axis_name="s")

    @functools.partial(
        pl.kernel,
        mesh=mesh,
        out_type=jax.ShapeDtypeStruct((n_out, D), x.dtype),
        scratch_types=[
            pltpu.VMEM((chunk,), jnp.int32),
            pltpu.VMEM((chunk, D), x.dtype),
            pltpu.SemaphoreType.DMA,
        ],
    )
    def k(x_hbm, idx_hbm, out_hbm, idx_v, rows_v, sem):
        wid = lax.axis_index("s") * info.num_cores + lax.axis_index("c")
        base = wid * b_per_w
        for c in range(nchunks):
            off = base + c * chunk
            pltpu.sync_copy(idx_hbm.at[pl.ds(off, chunk)], idx_v)
            pltpu.sync_copy(x_hbm.at[pl.ds(off, chunk)], rows_v)
            pltpu.async_copy(rows_v, out_hbm.at[idx_v], sem).wait()

    return k(x, idx)


_FB = 2048


def _ffn_block_kernel(be_ref, x_ref, w1_ref, b1_ref, w2_ref, b2_ref,
                      g_ref, bt_ref, o_ref, acc_ref, *, nf):
    f = pl.program_id(0)
    g = pl.program_id(1)
    sl = pl.ds(g * _T, _T)
    x = x_ref[...]
    h = jnp.dot(x, w1_ref[0], preferred_element_type=jnp.float32)
    h = h + b1_ref[0]
    h = 0.5 * h * (1.0 + lax.erf(h * (1.0 / math.sqrt(2.0))))
    y = jnp.dot(h, w2_ref[0], preferred_element_type=jnp.float32)

    @pl.when(f == 0)
    def _():
        acc_ref[sl, :] = (y + x + b2_ref[0]).astype(jnp.bfloat16)

    @pl.when(jnp.logical_and(f > 0, f < nf - 1))
    def _():
        acc_ref[sl, :] += y.astype(jnp.bfloat16)

    @pl.when(f == nf - 1)
    def _():
        yv = acc_ref[sl, :].astype(jnp.float32) + y
        mean = jnp.mean(yv, axis=1, keepdims=True)
        yc = yv - mean
        var = jnp.mean(yc * yc, axis=1, keepdims=True)
        o_ref[...] = yc * lax.rsqrt(var + _EPS) * g_ref[...] + bt_ref[...]


def _grouped_ffn(xs, W1, b1, W2, b2, gamma2d, beta2d, block_expert, G):
    N_PAD, D = xs.shape
    E, _, F = W1.shape
    nf = F // _FB
    grid_spec = pltpu.PrefetchScalarGridSpec(
        num_scalar_prefetch=1,
        grid=(nf, G),
        in_specs=[
            pl.BlockSpec((_T, D), lambda f, g, be: (g, 0)),
            pl.BlockSpec((1, D, _FB), lambda f, g, be: (be[g], 0, f)),
            pl.BlockSpec((1, 1, _FB), lambda f, g, be: (be[g], 0, f)),
            pl.BlockSpec((1, _FB, D), lambda f, g, be: (be[g], f, 0)),
            pl.BlockSpec((1, 1, D), lambda f, g, be: (be[g], 0, 0)),
            pl.BlockSpec((1, D), lambda f, g, be: (0, 0)),
            pl.BlockSpec((1, D), lambda f, g, be: (0, 0)),
        ],
        out_specs=pl.BlockSpec(
            (_T, D), lambda f, g, be: (jnp.where(f == nf - 1, g, 0), 0)),
        scratch_shapes=[pltpu.VMEM((N_PAD, D), jnp.bfloat16)],
    )
    return pl.pallas_call(
        functools.partial(_ffn_block_kernel, nf=nf),
        grid_spec=grid_spec,
        out_shape=jax.ShapeDtypeStruct((N_PAD, D), jnp.float32),
    )(block_expert, xs, W1, b1, W2, b2, gamma2d, beta2d)


def kernel(hidden_states, W1, b1, W2, b2, gamma, beta, expert_assign):
    B, S, D = hidden_states.shape
    E, _, F = W1.shape
    N = B * S
    G = N // _T + E
    N_PAD = G * _T

    x = hidden_states.reshape(N, D)
    e = expert_assign

    oh = (e[:, None] == jnp.arange(E, dtype=e.dtype)[None, :]).astype(jnp.int32)
    csum = jnp.cumsum(oh, axis=0)
    counts = csum[-1]
    rank = jnp.sum(csum * oh, axis=1) - 1
    blocks_e = (counts + _T - 1) // _T
    block_start = jnp.concatenate(
        [jnp.zeros((1,), jnp.int32), jnp.cumsum(blocks_e).astype(jnp.int32)])
    dest = block_start[e] * _T + rank
    gidx = jnp.arange(G, dtype=jnp.int32)
    block_expert = jnp.minimum(
        jnp.sum(block_start[1:E + 1][None, :] <= gidx[:, None], axis=1),
        E - 1).astype(jnp.int32)

    xs = _sc_scatter_rows(x, dest, N_PAD, chunk=64)

    zs = _grouped_ffn(xs, W1, b1.reshape(E, 1, F), W2, b2.reshape(E, 1, D),
                      gamma.reshape(1, D), beta.reshape(1, D),
                      block_expert, block_start[E])

    out = _sc_gather_rows(zs, dest, chunk=64)
    return out.reshape(B, S, D)
